# baseline (speedup 1.0000x reference)
import functools

import jax
import jax.numpy as jnp
from jax import lax
from jax.experimental import pallas as pl
from jax.experimental.pallas import tpu as pltpu
from jax.experimental.pallas import tpu_sc as plsc

N_TOKENS = 819200
K = 8
T_STEPS = 300

NC = 2
NS = 16
NW = NC * NS
L = 16

TOK_PER_W = N_TOKENS // NW
BLK = 320
NBLK = TOK_PER_W // BLK
GROUPS = BLK // L

USE_PHYS = False


def _sc_body(x_hbm, q_hbm, t_hbm, out_hbm, q_v, x_v, t_v, o_v):
    wid = lax.axis_index("s") * NC + lax.axis_index("c")

    pltpu.sync_copy(q_hbm, q_v)

    iota = lax.iota(jnp.int32, L)
    zeros = iota - iota
    phys0 = (iota // 8) * 1024 + (iota % 8) * 128

    def group(g, _):
        tt = t_v[pl.ds(g * L, L)]
        qbase = tt * (K * K)
        de = plsc.load_gather(q_v, [qbase])
        cc = plsc.load_gather(q_v, [qbase + K])
        dd = plsc.load_gather(q_v, [qbase + K + 1])
        if USE_PHYS:
            pb = g * (L * 128) + phys0

            def ld(j):
                return plsc.load_gather(x_v, [zeros, pb + j])

            def st(j, v):
                plsc.store_scatter(o_v, [zeros, pb + j], v)
        else:
            rows = g * L + iota

            def ld(j):
                return plsc.load_gather(x_v, [rows, zeros + j])

            def st(j, v):
                plsc.store_scatter(o_v, [rows, zeros + j], v)

        x0c = ld(0)
        s = ld(1)
        st(1, s * dd)
        for j in range(2, K):
            xj = ld(j)
            s = s + xj
            st(j, xj * dd)
        st(0, x0c * de + cc * s)
        return 0

    def block(b, _):
        base = wid * TOK_PER_W + b * BLK
        pltpu.sync_copy(x_hbm.at[pl.ds(base, BLK)], x_v)
        pltpu.sync_copy(t_hbm.at[pl.ds(base, BLK)], t_v)
        lax.fori_loop(0, GROUPS, group, 0)
        pltpu.sync_copy(o_v, out_hbm.at[pl.ds(base, BLK)])
        return 0

    lax.fori_loop(0, NBLK, block, 0)


_sc_call = functools.partial(
    pl.kernel,
    mesh=plsc.VectorSubcoreMesh(core_axis_name="c", subcore_axis_name="s"),
    out_type=jax.ShapeDtypeStruct((N_TOKENS, K), jnp.float32),
    scratch_types=[
        pltpu.VMEM((T_STEPS * K * K,), jnp.float32),
        pltpu.VMEM((BLK, K), jnp.float32),
        pltpu.VMEM((BLK,), jnp.int32),
        pltpu.VMEM((BLK, K), jnp.float32),
    ],
    compiler_params=pltpu.CompilerParams(needs_layout_passes=False),
)(_sc_body)


def kernel(x0, Qt_bar, t):
    return _sc_call(x0, Qt_bar.reshape(-1), t)


# speedup vs baseline: 1.4582x; 1.4582x over previous
import functools

import jax
import jax.numpy as jnp
from jax import lax
from jax.experimental import pallas as pl
from jax.experimental.pallas import tpu as pltpu
from jax.experimental.pallas import tpu_sc as plsc

N_TOKENS = 819200
K = 8
T_STEPS = 300

NC = 2
NS = 16
NW = NC * NS
L = 16

TOK_PER_W = N_TOKENS // NW
BLK = 160
NBLK = TOK_PER_W // BLK
GROUPS = BLK // L


def _sc_body(x_hbm, q_hbm, t_hbm, out_hbm,
             q_v, x_v0, x_v1, t_v0, t_v1, o_v0, o_v1,
             sin0, sin1, sout0, sout1):
    wid = lax.axis_index("s") * NC + lax.axis_index("c")
    x_v = (x_v0, x_v1)
    t_v = (t_v0, t_v1)
    o_v = (o_v0, o_v1)
    sin = (sin0, sin1)
    sout = (sout0, sout1)

    pltpu.sync_copy(q_hbm, q_v)

    iota = lax.iota(jnp.int32, L)
    zeros = iota - iota
    base0 = wid * TOK_PER_W

    def compute(k):
        tvk, xvk, ovk = t_v[k], x_v[k], o_v[k]

        def group(g, _):
            tt = tvk[pl.ds(g * L, L)]
            qbase = tt * (K * K)
            de = plsc.load_gather(q_v, [qbase])
            cc = plsc.load_gather(q_v, [qbase + K])
            dd = plsc.load_gather(q_v, [qbase + K + 1])
            rows = g * L + iota
            x0c = plsc.load_gather(xvk, [rows, zeros])
            s = plsc.load_gather(xvk, [rows, zeros + 1])
            plsc.store_scatter(ovk, [rows, zeros + 1], s * dd)
            for j in range(2, K):
                xj = plsc.load_gather(xvk, [rows, zeros + j])
                s = s + xj
                plsc.store_scatter(ovk, [rows, zeros + j], xj * dd)
            plsc.store_scatter(ovk, [rows, zeros], x0c * de + cc * s)
            return 0

        lax.fori_loop(0, GROUPS, group, 0)

    def fire_in(b, k):
        pltpu.async_copy(x_hbm.at[pl.ds(base0 + b * BLK, BLK)], x_v[k], sin[k])
        pltpu.async_copy(t_hbm.at[pl.ds(base0 + b * BLK, BLK)], t_v[k], sin[k])

    def drain_in(b, k):
        pltpu.make_async_copy(
            x_hbm.at[pl.ds(base0 + b * BLK, BLK)], x_v[k], sin[k]).wait()
        pltpu.make_async_copy(
            t_hbm.at[pl.ds(base0 + b * BLK, BLK)], t_v[k], sin[k]).wait()

    def fire_out(b, k):
        pltpu.async_copy(o_v[k], out_hbm.at[pl.ds(base0 + b * BLK, BLK)],
                         sout[k])

    def drain_out(b, k):
        pltpu.make_async_copy(
            o_v[k], out_hbm.at[pl.ds(base0 + b * BLK, BLK)], sout[k]).wait()

    fire_in(0, 0)
    fire_in(1, 1)

    def block_pair(bb, _):
        for k in range(2):
            b = bb * 2 + k

            @pl.when(b >= 2)
            def _():
                drain_out(b - 2, k)

            drain_in(b, k)
            compute(k)
            fire_out(b, k)

            @pl.when(b + 2 < NBLK)
            def _():
                fire_in(b + 2, k)

        return 0

    lax.fori_loop(0, NBLK // 2, block_pair, 0)
    drain_out(NBLK - 2, 0)
    drain_out(NBLK - 1, 1)


_sc_call = functools.partial(
    pl.kernel,
    mesh=plsc.VectorSubcoreMesh(core_axis_name="c", subcore_axis_name="s"),
    out_type=jax.ShapeDtypeStruct((N_TOKENS, K), jnp.float32),
    scratch_types=[
        pltpu.VMEM((T_STEPS * K * K,), jnp.float32),
        pltpu.VMEM((BLK, K), jnp.float32),
        pltpu.VMEM((BLK, K), jnp.float32),
        pltpu.VMEM((BLK,), jnp.int32),
        pltpu.VMEM((BLK,), jnp.int32),
        pltpu.VMEM((BLK, K), jnp.float32),
        pltpu.VMEM((BLK, K), jnp.float32),
        pltpu.SemaphoreType.DMA,
        pltpu.SemaphoreType.DMA,
        pltpu.SemaphoreType.DMA,
        pltpu.SemaphoreType.DMA,
    ],
    compiler_params=pltpu.CompilerParams(needs_layout_passes=False),
)(_sc_body)


def kernel(x0, Qt_bar, t):
    return _sc_call(x0, Qt_bar.reshape(-1), t)
